# Initial kernel scaffold; baseline (speedup 1.0000x reference)
import jax
import jax.numpy as jnp
from jax.experimental import pallas as pl


def kernel(protein_embeddings, edge_index, W_in, b_in, W_g, b_g, W1, b1, W2, b2, W3, b3):
    raise NotImplementedError("write your pallas kernel here")


# speedup vs baseline: 10.5562x; 10.5562x over previous
import functools

import jax
import jax.numpy as jnp
from jax import lax
from jax.experimental import pallas as pl
from jax.experimental.pallas import tpu as pltpu
from jax.experimental.pallas import tpu_sc as plsc

N_GENES = 512
IN_DIM = 1024
EMBED = 128
BATCH = 32
N_EDGES = 16384
HID1 = 1024
HID2 = 256


_NC = 2
_NS = 16
_E_PER = N_EDGES // (_NC * _NS)
_CHUNKS = _E_PER // 128
_CELLS = N_GENES * N_GENES
_CELLS_PER_TILE = _CELLS // _NS


def _edge_count_body(edges_hbm, out_hbm, src_v, dst_v, idx2, ones_v, zbuf,
                     shared_c):
    c = lax.axis_index("c")
    s = lax.axis_index("s")
    wid = c * _NS + s
    base = wid * _E_PER

    pltpu.sync_copy(edges_hbm.at[0, pl.ds(base, _E_PER)], src_v)
    pltpu.sync_copy(edges_hbm.at[1, pl.ds(base, _E_PER)], dst_v)

    for k in range(_E_PER // 16):
        sl = pl.ds(k * 16, 16)
        flat = dst_v[sl] * N_GENES + src_v[sl]
        idx2[k // 8, pl.ds((k % 8) * 16, 16)] = flat

    for j in range(8):
        ones_v[pl.ds(j * 16, 16)] = jnp.ones((16,), jnp.float32)

    def _zero(i, carry):
        zbuf[pl.ds(i * 16, 16)] = jnp.zeros((16,), jnp.float32)
        return carry
    lax.fori_loop(0, _CELLS_PER_TILE // 16, _zero, 0)
    pltpu.sync_copy(zbuf, shared_c.at[pl.ds(s * _CELLS_PER_TILE,
                                            _CELLS_PER_TILE)])
    plsc.subcore_barrier()

    for j in range(_CHUNKS):
        pltpu.sync_copy(ones_v, shared_c.at[idx2.at[j]], add=True)
    plsc.subcore_barrier()

    pltpu.sync_copy(
        shared_c.at[pl.ds(s * _CELLS_PER_TILE, _CELLS_PER_TILE)],
        out_hbm.at[c, pl.ds(s * _CELLS_PER_TILE, _CELLS_PER_TILE)])


@functools.cache
def _edge_counts_call():
    return functools.partial(
        pl.kernel,
        out_type=jax.ShapeDtypeStruct((_NC, _CELLS), jnp.float32),
        mesh=plsc.VectorSubcoreMesh(core_axis_name="c",
                                    subcore_axis_name="s"),
        scratch_types=[
            pltpu.VMEM((_E_PER,), jnp.int32),
            pltpu.VMEM((_E_PER,), jnp.int32),
            pltpu.VMEM((_CHUNKS, 128), jnp.int32),
            pltpu.VMEM((128,), jnp.float32),
            pltpu.VMEM((_CELLS_PER_TILE,), jnp.float32),
            pltpu.VMEM_SHARED((_CELLS,), jnp.float32),
        ],
    )(_edge_count_body)


_G1 = 8


def _embed_body(pe_ref, win_ref, bin_ref, wg_ref, out_ref):
    wg = wg_ref[...]
    for i in range(_G1):
        x = jnp.dot(pe_ref[i], win_ref[i], preferred_element_type=jnp.float32)
        x = x + bin_ref[i][None, :]
        x = jnp.where(x >= 0.0, x, 0.01 * x)
        out_ref[i] = jnp.dot(x, wg, preferred_element_type=jnp.float32)


_embed = pl.pallas_call(
    _embed_body,
    grid=(N_GENES // _G1,),
    in_specs=[
        pl.BlockSpec((_G1, BATCH, IN_DIM), lambda i: (i, 0, 0)),
        pl.BlockSpec((_G1, IN_DIM, EMBED), lambda i: (i, 0, 0)),
        pl.BlockSpec((_G1, EMBED), lambda i: (i, 0)),
        pl.BlockSpec((EMBED, EMBED), lambda i: (0, 0)),
    ],
    out_specs=pl.BlockSpec((_G1, BATCH, EMBED), lambda i: (i, 0, 0)),
    out_shape=jax.ShapeDtypeStruct((N_GENES, BATCH, EMBED), jnp.float32),
    compiler_params=pltpu.CompilerParams(
        dimension_semantics=("arbitrary",)),
)


_NB = 1024


def _agg_body(cp_ref, h_ref, bg_ref, out_ref, a_ref, dinv_ref):
    j = pl.program_id(0)

    @pl.when(j == 0)
    def _():
        csum = cp_ref[0] + cp_ref[1]
        deg = jnp.sum(csum, axis=1, keepdims=True) + 1.0
        dinv = 1.0 / jnp.sqrt(deg)
        dinv_ref[...] = dinv
        rows = lax.broadcasted_iota(jnp.int32, (N_GENES, N_GENES), 0)
        cols = lax.broadcasted_iota(jnp.int32, (N_GENES, N_GENES), 1)
        eye = jnp.where(rows == cols, 1.0, 0.0)
        a_ref[...] = (csum + eye) * dinv

    hs = h_ref[...] * dinv_ref[...]
    out_ref[...] = (
        jnp.dot(a_ref[...], hs, preferred_element_type=jnp.float32)
        + bg_ref[...])


_agg = pl.pallas_call(
    _agg_body,
    grid=(BATCH * EMBED // _NB,),
    in_specs=[
        pl.BlockSpec((_NC, N_GENES, N_GENES), lambda j: (0, 0, 0)),
        pl.BlockSpec((N_GENES, _NB), lambda j: (0, j)),
        pl.BlockSpec((1, _NB), lambda j: (0, j)),
    ],
    out_specs=pl.BlockSpec((N_GENES, _NB), lambda j: (0, j)),
    out_shape=jax.ShapeDtypeStruct((N_GENES, BATCH * EMBED), jnp.float32),
    scratch_shapes=[
        pltpu.VMEM((N_GENES, N_GENES), jnp.float32),
        pltpu.VMEM((N_GENES, 1), jnp.float32),
    ],
    compiler_params=pltpu.CompilerParams(
        dimension_semantics=("arbitrary",)),
)


_G3 = 8


def _mlp_body(agg_ref, w1_ref, b1_ref, w2_ref, b2_ref, w3_ref, b3_ref,
              out_ref, acc_ref):
    i = pl.program_id(0)

    @pl.when(i == 0)
    def _():
        acc_ref[...] = jnp.zeros_like(acc_ref)

    blk = agg_ref[...]
    x = jnp.transpose(blk, (1, 0, 2)).reshape(BATCH, _G3 * EMBED)
    acc_ref[...] += jnp.dot(x, w1_ref[...],
                            preferred_element_type=jnp.float32)

    @pl.when(i == pl.num_programs(0) - 1)
    def _():
        h1 = jnp.maximum(acc_ref[...] + b1_ref[...], 0.0)
        h2 = jnp.maximum(
            jnp.dot(h1, w2_ref[...], preferred_element_type=jnp.float32)
            + b2_ref[...], 0.0)
        out_ref[...] = (
            jnp.dot(h2, w3_ref[...], preferred_element_type=jnp.float32)
            + b3_ref[...])


_mlp = pl.pallas_call(
    _mlp_body,
    grid=(N_GENES // _G3,),
    in_specs=[
        pl.BlockSpec((_G3, BATCH, EMBED), lambda i: (i, 0, 0)),
        pl.BlockSpec((_G3 * EMBED, HID1), lambda i: (i, 0)),
        pl.BlockSpec((1, HID1), lambda i: (0, 0)),
        pl.BlockSpec((HID1, HID2), lambda i: (0, 0)),
        pl.BlockSpec((1, HID2), lambda i: (0, 0)),
        pl.BlockSpec((HID2, 1), lambda i: (0, 0)),
        pl.BlockSpec((1, 1), lambda i: (0, 0)),
    ],
    out_specs=pl.BlockSpec((BATCH, 1), lambda i: (0, 0)),
    out_shape=jax.ShapeDtypeStruct((BATCH, 1), jnp.float32),
    scratch_shapes=[pltpu.VMEM((BATCH, HID1), jnp.float32)],
    compiler_params=pltpu.CompilerParams(
        dimension_semantics=("arbitrary",)),
)


def kernel(protein_embeddings, edge_index, W_in, b_in, W_g, b_g,
           W1, b1, W2, b2, W3, b3):
    h = _embed(protein_embeddings, W_in, b_in, W_g)
    cp = _edge_counts_call()(edge_index)
    bg_tiled = jnp.tile(b_g, BATCH)[None, :]
    agg = _agg(cp.reshape(_NC, N_GENES, N_GENES),
               h.reshape(N_GENES, BATCH * EMBED), bg_tiled)
    preds = _mlp(agg.reshape(N_GENES, BATCH, EMBED),
                 W1, b1[None, :], W2, b2[None, :], W3, b3[None, :])
    return preds


# speedup vs baseline: 11.1498x; 1.0562x over previous
import functools

import jax
import jax.numpy as jnp
from jax import lax
from jax.experimental import pallas as pl
from jax.experimental.pallas import tpu as pltpu
from jax.experimental.pallas import tpu_sc as plsc

N_GENES = 512
IN_DIM = 1024
EMBED = 128
BATCH = 32
N_EDGES = 16384
HID1 = 1024
HID2 = 256


_NC = 2
_NS = 16
_E_PER = N_EDGES // (_NC * _NS)
_CHUNKS = _E_PER // 128
_CELLS = N_GENES * N_GENES
_CELLS_PER_TILE = _CELLS // _NS


def _edge_count_body(edges_hbm, out_hbm, src_v, dst_v, idx2, ones_v, zbuf,
                     shared_c):
    c = lax.axis_index("c")
    s = lax.axis_index("s")
    wid = c * _NS + s
    base = wid * _E_PER

    pltpu.sync_copy(edges_hbm.at[0, pl.ds(base, _E_PER)], src_v)
    pltpu.sync_copy(edges_hbm.at[1, pl.ds(base, _E_PER)], dst_v)

    for k in range(_E_PER // 16):
        sl = pl.ds(k * 16, 16)
        flat = dst_v[sl] * N_GENES + src_v[sl]
        idx2[k // 8, pl.ds((k % 8) * 16, 16)] = flat

    for j in range(8):
        ones_v[pl.ds(j * 16, 16)] = jnp.ones((16,), jnp.float32)

    def _zero(i, carry):
        zbuf[pl.ds(i * 16, 16)] = jnp.zeros((16,), jnp.float32)
        return carry
    lax.fori_loop(0, _CELLS_PER_TILE // 16, _zero, 0)
    pltpu.sync_copy(zbuf, shared_c.at[pl.ds(s * _CELLS_PER_TILE,
                                            _CELLS_PER_TILE)])
    plsc.subcore_barrier()

    for j in range(_CHUNKS):
        pltpu.sync_copy(ones_v, shared_c.at[idx2.at[j]], add=True)
    plsc.subcore_barrier()

    pltpu.sync_copy(
        shared_c.at[pl.ds(s * _CELLS_PER_TILE, _CELLS_PER_TILE)],
        out_hbm.at[c, pl.ds(s * _CELLS_PER_TILE, _CELLS_PER_TILE)])


@functools.cache
def _edge_counts_call():
    return functools.partial(
        pl.kernel,
        out_type=jax.ShapeDtypeStruct((_NC, _CELLS), jnp.float32),
        mesh=plsc.VectorSubcoreMesh(core_axis_name="c",
                                    subcore_axis_name="s"),
        scratch_types=[
            pltpu.VMEM((_E_PER,), jnp.int32),
            pltpu.VMEM((_E_PER,), jnp.int32),
            pltpu.VMEM((_CHUNKS, 128), jnp.int32),
            pltpu.VMEM((128,), jnp.float32),
            pltpu.VMEM((_CELLS_PER_TILE,), jnp.float32),
            pltpu.VMEM_SHARED((_CELLS,), jnp.float32),
        ],
    )(_edge_count_body)


_G1 = 8


def _embed_body(pe_ref, win_ref, bin_ref, wg_ref, out_ref):
    wg = wg_ref[...]
    for i in range(_G1):
        x = jnp.dot(pe_ref[i], win_ref[i], preferred_element_type=jnp.float32)
        x = x + bin_ref[i][None, :]
        x = jnp.where(x >= 0.0, x, 0.01 * x)
        out_ref[i] = jnp.dot(x, wg, preferred_element_type=jnp.float32)


_embed = pl.pallas_call(
    _embed_body,
    grid=(N_GENES // _G1,),
    in_specs=[
        pl.BlockSpec((_G1, BATCH, IN_DIM), lambda i: (i, 0, 0)),
        pl.BlockSpec((_G1, IN_DIM, EMBED), lambda i: (i, 0, 0)),
        pl.BlockSpec((_G1, EMBED), lambda i: (i, 0)),
        pl.BlockSpec((EMBED, EMBED), lambda i: (0, 0)),
    ],
    out_specs=pl.BlockSpec((_G1, BATCH, EMBED), lambda i: (i, 0, 0)),
    out_shape=jax.ShapeDtypeStruct((N_GENES, BATCH, EMBED), jnp.float32),
    compiler_params=pltpu.CompilerParams(
        dimension_semantics=("arbitrary",)),
)


_G3 = 8


def _mlp_body(cp_ref, h_ref, bg_ref, w1_ref, b1_ref, w2_ref, b2_ref,
              w3_ref, b3_ref, out_ref, agg_ref, acc_ref):
    i = pl.program_id(0)

    @pl.when(i == 0)
    def _():
        csum = cp_ref[0] + cp_ref[1]
        deg = jnp.sum(csum, axis=1, keepdims=True) + 1.0
        dinv = 1.0 / jnp.sqrt(deg)
        rows = lax.broadcasted_iota(jnp.int32, (N_GENES, N_GENES), 0)
        cols = lax.broadcasted_iota(jnp.int32, (N_GENES, N_GENES), 1)
        eye = jnp.where(rows == cols, 1.0, 0.0)
        m = (csum + eye) * dinv
        a = jnp.dot(m, eye * dinv, preferred_element_type=jnp.float32)
        agg_ref[...] = (
            jnp.dot(a, h_ref[...], preferred_element_type=jnp.float32)
            + bg_ref[...])
        acc_ref[...] = jnp.zeros_like(acc_ref)

    blk = agg_ref[pl.ds(i * _G3, _G3), :].reshape(_G3, BATCH, EMBED)
    x = jnp.transpose(blk, (1, 0, 2)).reshape(BATCH, _G3 * EMBED)
    acc_ref[...] += jnp.dot(x, w1_ref[...],
                            preferred_element_type=jnp.float32)

    @pl.when(i == pl.num_programs(0) - 1)
    def _():
        h1 = jnp.maximum(acc_ref[...] + b1_ref[...], 0.0)
        h2 = jnp.maximum(
            jnp.dot(h1, w2_ref[...], preferred_element_type=jnp.float32)
            + b2_ref[...], 0.0)
        out_ref[...] = (
            jnp.dot(h2, w3_ref[...], preferred_element_type=jnp.float32)
            + b3_ref[...])


_mlp = pl.pallas_call(
    _mlp_body,
    grid=(N_GENES // _G3,),
    in_specs=[
        pl.BlockSpec((_NC, N_GENES, N_GENES), lambda i: (0, 0, 0)),
        pl.BlockSpec((N_GENES, BATCH * EMBED), lambda i: (0, 0)),
        pl.BlockSpec((1, BATCH * EMBED), lambda i: (0, 0)),
        pl.BlockSpec((_G3 * EMBED, HID1), lambda i: (i, 0)),
        pl.BlockSpec((1, HID1), lambda i: (0, 0)),
        pl.BlockSpec((HID1, HID2), lambda i: (0, 0)),
        pl.BlockSpec((1, HID2), lambda i: (0, 0)),
        pl.BlockSpec((HID2, 1), lambda i: (0, 0)),
        pl.BlockSpec((1, 1), lambda i: (0, 0)),
    ],
    out_specs=pl.BlockSpec((BATCH, 1), lambda i: (0, 0)),
    out_shape=jax.ShapeDtypeStruct((BATCH, 1), jnp.float32),
    scratch_shapes=[
        pltpu.VMEM((N_GENES, BATCH * EMBED), jnp.float32),
        pltpu.VMEM((BATCH, HID1), jnp.float32),
    ],
    compiler_params=pltpu.CompilerParams(
        dimension_semantics=("arbitrary",)),
)


def kernel(protein_embeddings, edge_index, W_in, b_in, W_g, b_g,
           W1, b1, W2, b2, W3, b3):
    h = _embed(protein_embeddings, W_in, b_in, W_g)
    cp = _edge_counts_call()(edge_index)
    bg_tiled = jnp.tile(b_g, BATCH)[None, :]
    preds = _mlp(cp.reshape(_NC, N_GENES, N_GENES),
                 h.reshape(N_GENES, BATCH * EMBED), bg_tiled,
                 W1, b1[None, :], W2, b2[None, :], W3, b3[None, :])
    return preds


# speedup vs baseline: 12.6158x; 1.1315x over previous
import functools

import jax
import jax.numpy as jnp
from jax import lax
from jax.experimental import pallas as pl
from jax.experimental.pallas import tpu as pltpu
from jax.experimental.pallas import tpu_sc as plsc

N_GENES = 512
IN_DIM = 1024
EMBED = 128
BATCH = 32
N_EDGES = 16384
HID1 = 1024
HID2 = 256


_NC = 2
_NS = 16
_E_PER = N_EDGES // (_NC * _NS)
_CHUNKS = _E_PER // 128
_CELLS = N_GENES * N_GENES
_CELLS_PER_TILE = _CELLS // _NS


def _edge_count_body(edges_hbm, out_hbm, src_v, dst_v, idx2, ones_v, zbuf,
                     shared_c):
    c = lax.axis_index("c")
    s = lax.axis_index("s")
    wid = c * _NS + s
    base = wid * _E_PER

    pltpu.sync_copy(edges_hbm.at[0, pl.ds(base, _E_PER)], src_v)
    pltpu.sync_copy(edges_hbm.at[1, pl.ds(base, _E_PER)], dst_v)

    for k in range(_E_PER // 16):
        sl = pl.ds(k * 16, 16)
        flat = dst_v[sl] * N_GENES + src_v[sl]
        idx2[k // 8, pl.ds((k % 8) * 16, 16)] = flat

    for j in range(8):
        ones_v[pl.ds(j * 16, 16)] = jnp.ones((16,), jnp.float32)

    def _zero(i, carry):
        zbuf[pl.ds(i * 16, 16)] = jnp.zeros((16,), jnp.float32)
        return carry
    lax.fori_loop(0, _CELLS_PER_TILE // 16, _zero, 0)
    pltpu.sync_copy(zbuf, shared_c.at[pl.ds(s * _CELLS_PER_TILE,
                                            _CELLS_PER_TILE)])
    plsc.subcore_barrier()

    for j in range(_CHUNKS):
        pltpu.sync_copy(ones_v, shared_c.at[idx2.at[j]], add=True)
    plsc.subcore_barrier()

    pltpu.sync_copy(
        shared_c.at[pl.ds(s * _CELLS_PER_TILE, _CELLS_PER_TILE)],
        out_hbm.at[c, pl.ds(s * _CELLS_PER_TILE, _CELLS_PER_TILE)])


@functools.cache
def _edge_counts_call():
    return functools.partial(
        pl.kernel,
        out_type=jax.ShapeDtypeStruct((_NC, _CELLS), jnp.float32),
        mesh=plsc.VectorSubcoreMesh(core_axis_name="c",
                                    subcore_axis_name="s"),
        scratch_types=[
            pltpu.VMEM((_E_PER,), jnp.int32),
            pltpu.VMEM((_E_PER,), jnp.int32),
            pltpu.VMEM((_CHUNKS, 128), jnp.int32),
            pltpu.VMEM((128,), jnp.float32),
            pltpu.VMEM((_CELLS_PER_TILE,), jnp.float32),
            pltpu.VMEM_SHARED((_CELLS,), jnp.float32),
        ],
    )(_edge_count_body)


_G1 = 16


def _embed_body(pe_ref, win_ref, bin_ref, wg_ref, out_ref):
    wg = wg_ref[...]
    for i in range(_G1):
        x = jnp.dot(pe_ref[i], win_ref[i], preferred_element_type=jnp.float32)
        x = x + bin_ref[i][None, :]
        x = jnp.where(x >= 0.0, x, 0.01 * x)
        out_ref[i] = jnp.dot(x, wg, preferred_element_type=jnp.float32)


_embed = pl.pallas_call(
    _embed_body,
    grid=(N_GENES // _G1,),
    in_specs=[
        pl.BlockSpec((_G1, BATCH, IN_DIM), lambda i: (i, 0, 0)),
        pl.BlockSpec((_G1, IN_DIM, EMBED), lambda i: (i, 0, 0)),
        pl.BlockSpec((_G1, EMBED), lambda i: (i, 0)),
        pl.BlockSpec((EMBED, EMBED), lambda i: (0, 0)),
    ],
    out_specs=pl.BlockSpec((_G1, BATCH, EMBED), lambda i: (i, 0, 0)),
    out_shape=jax.ShapeDtypeStruct((N_GENES, BATCH, EMBED), jnp.float32),
    compiler_params=pltpu.CompilerParams(
        dimension_semantics=("arbitrary",)),
)


_G3 = 16


def _mlp_body(cp_ref, h_ref, bg_ref, w1_ref, b1_ref, w2_ref, b2_ref,
              w3_ref, b3_ref, out_ref, agg_ref, acc_ref):
    i = pl.program_id(0)

    @pl.when(i == 0)
    def _():
        csum = cp_ref[0] + cp_ref[1]
        deg = jnp.sum(csum, axis=1, keepdims=True) + 1.0
        dinv = 1.0 / jnp.sqrt(deg)
        rows = lax.broadcasted_iota(jnp.int32, (N_GENES, N_GENES), 0)
        cols = lax.broadcasted_iota(jnp.int32, (N_GENES, N_GENES), 1)
        eye = jnp.where(rows == cols, 1.0, 0.0)
        m = (csum + eye) * dinv
        a = jnp.dot(m, eye * dinv, preferred_element_type=jnp.float32)
        agg_ref[...] = (
            jnp.dot(a, h_ref[...], preferred_element_type=jnp.float32)
            + bg_ref[...])
        acc_ref[...] = jnp.zeros_like(acc_ref)

    blk = agg_ref[pl.ds(i * _G3, _G3), :].reshape(_G3, BATCH, EMBED)
    x = jnp.transpose(blk, (1, 0, 2)).reshape(BATCH, _G3 * EMBED)
    acc_ref[...] += jnp.dot(x, w1_ref[...],
                            preferred_element_type=jnp.float32)

    @pl.when(i == pl.num_programs(0) - 1)
    def _():
        h1 = jnp.maximum(acc_ref[...] + b1_ref[...], 0.0)
        h2 = jnp.maximum(
            jnp.dot(h1, w2_ref[...], preferred_element_type=jnp.float32)
            + b2_ref[...], 0.0)
        out_ref[...] = (
            jnp.dot(h2, w3_ref[...], preferred_element_type=jnp.float32)
            + b3_ref[...])


_mlp = pl.pallas_call(
    _mlp_body,
    grid=(N_GENES // _G3,),
    in_specs=[
        pl.BlockSpec((_NC, N_GENES, N_GENES), lambda i: (0, 0, 0)),
        pl.BlockSpec((N_GENES, BATCH * EMBED), lambda i: (0, 0)),
        pl.BlockSpec((1, BATCH * EMBED), lambda i: (0, 0)),
        pl.BlockSpec((_G3 * EMBED, HID1), lambda i: (i, 0)),
        pl.BlockSpec((1, HID1), lambda i: (0, 0)),
        pl.BlockSpec((HID1, HID2), lambda i: (0, 0)),
        pl.BlockSpec((1, HID2), lambda i: (0, 0)),
        pl.BlockSpec((HID2, 1), lambda i: (0, 0)),
        pl.BlockSpec((1, 1), lambda i: (0, 0)),
    ],
    out_specs=pl.BlockSpec((BATCH, 1), lambda i: (0, 0)),
    out_shape=jax.ShapeDtypeStruct((BATCH, 1), jnp.float32),
    scratch_shapes=[
        pltpu.VMEM((N_GENES, BATCH * EMBED), jnp.float32),
        pltpu.VMEM((BATCH, HID1), jnp.float32),
    ],
    compiler_params=pltpu.CompilerParams(
        dimension_semantics=("arbitrary",)),
)


def kernel(protein_embeddings, edge_index, W_in, b_in, W_g, b_g,
           W1, b1, W2, b2, W3, b3):
    h = _embed(protein_embeddings, W_in, b_in, W_g)
    cp = _edge_counts_call()(edge_index)
    bg_tiled = jnp.tile(b_g, BATCH)[None, :]
    preds = _mlp(cp.reshape(_NC, N_GENES, N_GENES),
                 h.reshape(N_GENES, BATCH * EMBED), bg_tiled,
                 W1, b1[None, :], W2, b2[None, :], W3, b3[None, :])
    return preds


# speedup vs baseline: 12.8424x; 1.0180x over previous
import functools

import jax
import jax.numpy as jnp
from jax import lax
from jax.experimental import pallas as pl
from jax.experimental.pallas import tpu as pltpu
from jax.experimental.pallas import tpu_sc as plsc

N_GENES = 512
IN_DIM = 1024
EMBED = 128
BATCH = 32
N_EDGES = 16384
HID1 = 1024
HID2 = 256


_NC = 2
_NS = 16
_E_PER = N_EDGES // (_NC * _NS)
_CHUNKS = _E_PER // 128
_CELLS = N_GENES * N_GENES
_CELLS_PER_TILE = _CELLS // _NS


def _edge_count_body(edges_hbm, out_hbm, src_v, dst_v, idx2, ones_v, zbuf,
                     shared_c):
    c = lax.axis_index("c")
    s = lax.axis_index("s")
    wid = c * _NS + s
    base = wid * _E_PER

    pltpu.sync_copy(edges_hbm.at[0, pl.ds(base, _E_PER)], src_v)
    pltpu.sync_copy(edges_hbm.at[1, pl.ds(base, _E_PER)], dst_v)

    for k in range(_E_PER // 16):
        sl = pl.ds(k * 16, 16)
        flat = dst_v[sl] * N_GENES + src_v[sl]
        idx2[k // 8, pl.ds((k % 8) * 16, 16)] = flat

    for j in range(8):
        ones_v[pl.ds(j * 16, 16)] = jnp.ones((16,), jnp.float32)

    def _zero(i, carry):
        zbuf[pl.ds(i * 16, 16)] = jnp.zeros((16,), jnp.float32)
        return carry
    lax.fori_loop(0, _CELLS_PER_TILE // 16, _zero, 0)
    pltpu.sync_copy(zbuf, shared_c.at[pl.ds(s * _CELLS_PER_TILE,
                                            _CELLS_PER_TILE)])
    plsc.subcore_barrier()

    for j in range(_CHUNKS):
        pltpu.sync_copy(ones_v, shared_c.at[idx2.at[j]], add=True)
    plsc.subcore_barrier()

    pltpu.sync_copy(
        shared_c.at[pl.ds(s * _CELLS_PER_TILE, _CELLS_PER_TILE)],
        out_hbm.at[c, pl.ds(s * _CELLS_PER_TILE, _CELLS_PER_TILE)])


@functools.cache
def _edge_counts_call():
    return functools.partial(
        pl.kernel,
        out_type=jax.ShapeDtypeStruct((_NC, _CELLS), jnp.float32),
        mesh=plsc.VectorSubcoreMesh(core_axis_name="c",
                                    subcore_axis_name="s"),
        scratch_types=[
            pltpu.VMEM((_E_PER,), jnp.int32),
            pltpu.VMEM((_E_PER,), jnp.int32),
            pltpu.VMEM((_CHUNKS, 128), jnp.int32),
            pltpu.VMEM((128,), jnp.float32),
            pltpu.VMEM((_CELLS_PER_TILE,), jnp.float32),
            pltpu.VMEM_SHARED((_CELLS,), jnp.float32),
        ],
    )(_edge_count_body)


_G1 = 32


def _embed_body(pe_ref, win_ref, bin_ref, wg_ref, out_ref):
    wg = wg_ref[...]
    for i in range(_G1):
        x = jnp.dot(pe_ref[i], win_ref[i], preferred_element_type=jnp.float32)
        x = x + bin_ref[i][None, :]
        x = jnp.where(x >= 0.0, x, 0.01 * x)
        out_ref[i] = jnp.dot(x, wg, preferred_element_type=jnp.float32)


_embed = pl.pallas_call(
    _embed_body,
    grid=(N_GENES // _G1,),
    in_specs=[
        pl.BlockSpec((_G1, BATCH, IN_DIM), lambda i: (i, 0, 0)),
        pl.BlockSpec((_G1, IN_DIM, EMBED), lambda i: (i, 0, 0)),
        pl.BlockSpec((_G1, EMBED), lambda i: (i, 0)),
        pl.BlockSpec((EMBED, EMBED), lambda i: (0, 0)),
    ],
    out_specs=pl.BlockSpec((_G1, BATCH, EMBED), lambda i: (i, 0, 0)),
    out_shape=jax.ShapeDtypeStruct((N_GENES, BATCH, EMBED), jnp.float32),
    compiler_params=pltpu.CompilerParams(
        dimension_semantics=("arbitrary",)),
)


_G3 = 32


def _mlp_body(cp_ref, h_ref, bg_ref, w1_ref, b1_ref, w2_ref, b2_ref,
              w3_ref, b3_ref, out_ref, agg_ref, acc_ref):
    i = pl.program_id(0)

    @pl.when(i == 0)
    def _():
        csum = cp_ref[0] + cp_ref[1]
        deg = jnp.sum(csum, axis=1, keepdims=True) + 1.0
        dinv = 1.0 / jnp.sqrt(deg)
        rows = lax.broadcasted_iota(jnp.int32, (N_GENES, N_GENES), 0)
        cols = lax.broadcasted_iota(jnp.int32, (N_GENES, N_GENES), 1)
        eye = jnp.where(rows == cols, 1.0, 0.0)
        m = (csum + eye) * dinv
        a = jnp.dot(m, eye * dinv, preferred_element_type=jnp.float32)
        agg_ref[...] = (
            jnp.dot(a, h_ref[...], preferred_element_type=jnp.float32)
            + bg_ref[...])
        acc_ref[...] = jnp.zeros_like(acc_ref)

    blk = agg_ref[pl.ds(i * _G3, _G3), :].reshape(_G3, BATCH, EMBED)
    x = jnp.transpose(blk, (1, 0, 2)).reshape(BATCH, _G3 * EMBED)
    acc_ref[...] += jnp.dot(x, w1_ref[...],
                            preferred_element_type=jnp.float32)

    @pl.when(i == pl.num_programs(0) - 1)
    def _():
        h1 = jnp.maximum(acc_ref[...] + b1_ref[...], 0.0)
        h2 = jnp.maximum(
            jnp.dot(h1, w2_ref[...], preferred_element_type=jnp.float32)
            + b2_ref[...], 0.0)
        out_ref[...] = (
            jnp.dot(h2, w3_ref[...], preferred_element_type=jnp.float32)
            + b3_ref[...])


_mlp = pl.pallas_call(
    _mlp_body,
    grid=(N_GENES // _G3,),
    in_specs=[
        pl.BlockSpec((_NC, N_GENES, N_GENES), lambda i: (0, 0, 0)),
        pl.BlockSpec((N_GENES, BATCH * EMBED), lambda i: (0, 0)),
        pl.BlockSpec((1, BATCH * EMBED), lambda i: (0, 0)),
        pl.BlockSpec((_G3 * EMBED, HID1), lambda i: (i, 0)),
        pl.BlockSpec((1, HID1), lambda i: (0, 0)),
        pl.BlockSpec((HID1, HID2), lambda i: (0, 0)),
        pl.BlockSpec((1, HID2), lambda i: (0, 0)),
        pl.BlockSpec((HID2, 1), lambda i: (0, 0)),
        pl.BlockSpec((1, 1), lambda i: (0, 0)),
    ],
    out_specs=pl.BlockSpec((BATCH, 1), lambda i: (0, 0)),
    out_shape=jax.ShapeDtypeStruct((BATCH, 1), jnp.float32),
    scratch_shapes=[
        pltpu.VMEM((N_GENES, BATCH * EMBED), jnp.float32),
        pltpu.VMEM((BATCH, HID1), jnp.float32),
    ],
    compiler_params=pltpu.CompilerParams(
        dimension_semantics=("arbitrary",)),
)


def kernel(protein_embeddings, edge_index, W_in, b_in, W_g, b_g,
           W1, b1, W2, b2, W3, b3):
    h = _embed(protein_embeddings, W_in, b_in, W_g)
    cp = _edge_counts_call()(edge_index)
    bg_tiled = jnp.tile(b_g, BATCH)[None, :]
    preds = _mlp(cp.reshape(_NC, N_GENES, N_GENES),
                 h.reshape(N_GENES, BATCH * EMBED), bg_tiled,
                 W1, b1[None, :], W2, b2[None, :], W3, b3[None, :])
    return preds
